# baseline (speedup 1.0000x reference)
import functools

import jax
import jax.numpy as jnp
from jax import lax
from jax.experimental import pallas as pl
from jax.experimental.pallas import tpu as pltpu
from jax.experimental.pallas import tpu_sc as plsc

NC = 2
NS = 16
NW = NC * NS

VOCAB = 1_000_000
DIM = 32
BN = 4096
TN = 200

NBLK = 7812
MAIN_U = 244
TAILV = NBLK * 128
SROWS = VOCAB // 4

_mesh = plsc.VectorSubcoreMesh(core_axis_name="c", subcore_axis_name="s")
_tc_tiled = pltpu.CompilerParams(use_tc_tiling_on_sc=True)
_sc_tiled = pltpu.CompilerParams(use_tc_tiling_on_sc=False,
                                needs_layout_passes=False)

_IOTA = lambda: lax.iota(jnp.int32, 16)


def _c16(v):
    return jnp.full((16,), v, jnp.int32)


@functools.partial(
    pl.kernel,
    out_type=jax.ShapeDtypeStruct((NBLK * 32, 128), jnp.float32),
    mesh=_mesh,
    scratch_types=[
        pltpu.VMEM((32, 128), jnp.float32),
        pltpu.VMEM((32, 128), jnp.float32),
        pltpu.SemaphoreType.DMA,
        pltpu.SemaphoreType.DMA,
        pltpu.SemaphoreType.DMA,
        pltpu.SemaphoreType.DMA,
    ],
    compiler_params=_tc_tiled,
)
def _detile(tt_hbm, d_hbm, buf0, buf1, sr0, sr1, sw0, sw1):
    w = lax.axis_index("s") * NC + lax.axis_index("c")
    bufs, srs, sws = (buf0, buf1), (sr0, sr1), (sw0, sw1)

    def read(c, b):
        pltpu.async_copy(tt_hbm.at[:, pl.ds(c * 128, 128)], bufs[b], srs[b])

    read(w, 0)

    def step(u, carry):
        c = u * 32 + w
        for b in range(2):
            @pl.when(u % 2 == b)
            def _():
                @pl.when(jnp.logical_and(u >= 1, u + 1 < MAIN_U))
                def _():
                    pltpu.make_async_copy(
                        bufs[1 - b], d_hbm.at[pl.ds(0, 32), :],
                        sws[1 - b]).wait()
                @pl.when(u + 1 < MAIN_U)
                def _():
                    read(c + 32, 1 - b)
                pltpu.make_async_copy(
                    tt_hbm.at[:, pl.ds(c * 128, 128)], bufs[b], srs[b]).wait()
                pltpu.async_copy(
                    bufs[b], d_hbm.at[pl.ds(c * 32, 32), :], sws[b])
        return carry

    lax.fori_loop(0, MAIN_U, step, 0)
    for b in range(2):
        @pl.when((MAIN_U - 1) % 2 >= b)
        def _():
            pltpu.make_async_copy(
                bufs[b], d_hbm.at[pl.ds(0, 32), :], sws[b]).wait()

    @pl.when(w < NBLK - MAIN_U * 32)
    def _():
        c = MAIN_U * 32 + w
        pltpu.sync_copy(tt_hbm.at[:, pl.ds(c * 128, 128)], buf0)
        pltpu.sync_copy(buf0, d_hbm.at[pl.ds(c * 32, 32), :])


def _transpose_block(src, dst):
    for rp in range(32):
        for k in range(8):
            vals = plsc.load_gather(
                src, [16 * (k % 2) + _IOTA(), _c16(4 * rp + k // 2)])
            dst[rp, pl.ds(16 * k, 16)] = vals


@functools.partial(
    pl.kernel,
    out_type=jax.ShapeDtypeStruct((SROWS, 128), jnp.float32),
    mesh=_mesh,
    scratch_types=[
        pltpu.VMEM((32, 128), jnp.float32),
        pltpu.VMEM((32, 128), jnp.float32),
        pltpu.VMEM((32, 128), jnp.float32),
        pltpu.VMEM((32, 128), jnp.float32),
        pltpu.SemaphoreType.DMA,
        pltpu.SemaphoreType.DMA,
        pltpu.SemaphoreType.DMA,
        pltpu.SemaphoreType.DMA,
    ],
    compiler_params=_sc_tiled,
)
def _format_table(d_hbm, tails_hbm, s_hbm, in0, in1, out0, out1,
                  si0, si1, so0, so1):
    w = lax.axis_index("s") * NC + lax.axis_index("c")
    ins, outs = (in0, in1), (out0, out1)
    sis, sos = (si0, si1), (so0, so1)

    def read(c, b):
        pltpu.async_copy(d_hbm.at[pl.ds(c * 32, 32), :], ins[b], sis[b])

    read(w, 0)

    def step(u, carry):
        c = u * 32 + w
        for b in range(2):
            @pl.when(u % 2 == b)
            def _():
                @pl.when(u + 1 < MAIN_U)
                def _():
                    read(c + 32, 1 - b)
                pltpu.make_async_copy(
                    d_hbm.at[pl.ds(c * 32, 32), :], ins[b], sis[b]).wait()
                @pl.when(u >= 2)
                def _():
                    pltpu.make_async_copy(
                        outs[b], s_hbm.at[pl.ds(0, 32), :], sos[b]).wait()
                _transpose_block(ins[b], outs[b])
                pltpu.async_copy(
                    outs[b], s_hbm.at[pl.ds(c * 32, 32), :], sos[b])
        return carry

    lax.fori_loop(0, MAIN_U, step, 0)
    for b in range(2):
        @pl.when((MAIN_U - 1) % 2 >= b)
        def _():
            pltpu.make_async_copy(
                outs[b], s_hbm.at[pl.ds(0, 32), :], sos[b]).wait()

    @pl.when(w < NBLK - MAIN_U * 32)
    def _():
        c = MAIN_U * 32 + w
        pltpu.sync_copy(d_hbm.at[pl.ds(c * 32, 32), :], in0)
        _transpose_block(in0, out0)
        pltpu.sync_copy(out0, s_hbm.at[pl.ds(c * 32, 32), :])

    @pl.when(w == NW - 1)
    def _():
        buf16 = in1.at[pl.ds(0, 16), :]
        pltpu.sync_copy(tails_hbm, buf16)
        pltpu.sync_copy(buf16, s_hbm.at[pl.ds(NBLK * 32, 16), :])


@functools.partial(
    pl.kernel,
    out_type=jax.ShapeDtypeStruct((TN, 4, 32, 8, 128), jnp.float32),
    mesh=_mesh,
    scratch_types=[
        pltpu.VMEM((TN, 128), jnp.int32),
        pltpu.VMEM((128, DIM), jnp.float32),
        pltpu.VMEM((128, DIM), jnp.float32),
        pltpu.VMEM((32, 128), jnp.float32),
        pltpu.VMEM((32, 128), jnp.float32),
        pltpu.SemaphoreType.DMA,
        pltpu.SemaphoreType.DMA,
        pltpu.SemaphoreType.DMA,
        pltpu.SemaphoreType.DMA,
    ],
    compiler_params=_sc_tiled,
)
def _gather(s_hbm, xt_hbm, p_hbm, idx_v, g0, g1, o0, o1, sg0, sg1, so0, so1):
    w = lax.axis_index("s") * NC + lax.axis_index("c")
    b0 = w * 128
    gs, os = (g0, g1), (o0, o1)
    sgs, sos = (sg0, sg1), (so0, so1)

    pltpu.sync_copy(xt_hbm.at[:, pl.ds(b0, 128)], idx_v)

    def fire(t, b):
        pltpu.async_copy(s_hbm.at[idx_v.at[t]], gs[b], sgs[b])

    fire(0, 0)

    def write(t, b):
        for tr in range(4):
            pltpu.async_copy(
                os[b].at[pl.ds(8 * tr, 8), :],
                p_hbm.at[t, tr, w, :, :], sos[b])

    def wait_write(b):
        for tr in range(4):
            pltpu.make_async_copy(
                os[b].at[pl.ds(8 * tr, 8), :],
                p_hbm.at[0, 0, 0, :, :], sos[b]).wait()

    def step(t, carry):
        for b in range(2):
            @pl.when(t % 2 == b)
            def _():
                @pl.when(t + 1 < TN)
                def _():
                    fire(t + 1, 1 - b)
                pltpu.make_async_copy(
                    s_hbm.at[idx_v.at[t]], gs[b], sgs[b]).wait()
                @pl.when(t >= 2)
                def _():
                    wait_write(b)
                for k in range(8):
                    rows = 16 * k + _IOTA()
                    for d in range(DIM):
                        os[b][d, pl.ds(16 * k, 16)] = plsc.load_gather(
                            gs[b], [rows, _c16(d)])
                write(t, b)
        return carry

    lax.fori_loop(0, TN, step, 0)
    for b in range(2):
        wait_write(b)


def kernel(X, table):
    tail_s = table[TAILV:].reshape(16, 128)
    d = _detile(table.T)
    s = _format_table(d, tail_s)
    p = _gather(s.reshape(VOCAB, DIM), X.T)
    return p.transpose(2, 4, 0, 1, 3).reshape(BN, TN, DIM)
